# baseline (speedup 1.0000x reference)
import jax
import jax.numpy as jnp
from jax import lax
from jax.experimental import pallas as pl
from jax.experimental.pallas import tpu as pltpu
from jax.experimental.pallas import tpu_sc as plsc

_R = 1024
_N = 100000
_L = 16
_NC = 2
_NS = 16
_NW = _NC * _NS
_SCR = 512
_TCR = _R - _SCR
_GPW = _SCR // (_NW * 8)
_TCB = 8
_NFULL = _N // _L // 8 * 8 * _L

_CW = 3200
_NCH = 31
_MAIN = _NCH * _CW
_TW = 1024
_CVEC = _CW // _L
_TVEC = _TW // _L
_CAND = 128
_TOP_K = 5
_MIN_ACT = 0.5
_NEG = float(-jnp.inf)


def _sc_body(x_hbm, tail_hbm, cand_hbm, cb, tb, tops_buf, out_buf,
             sem0, sem1, semt):
    cid = lax.axis_index("c")
    sid = lax.axis_index("s")
    wid = cid * _NS + sid
    neg = jnp.full((_L,), _NEG, jnp.float32)
    sems = (sem0, sem1)

    def chunk_copy(grp, c, par, sem):
        return pltpu.make_async_copy(
            x_hbm.at[pl.ds(grp * 8, 8), pl.ds(c * _CW, _CW)],
            cb.at[par], sem)

    def scan_quad(buf, par, i, nvec, tops):
        def vec_step(j, t):
            t = list(t)
            xs = [buf[par, i + q, pl.ds(j * _L, _L)] for q in range(4)]
            for lev in range(5):
                for q in range(4):
                    m = jnp.maximum(t[lev * 4 + q], xs[q])
                    if lev < 4:
                        xs[q] = jnp.minimum(t[lev * 4 + q], xs[q])
                    t[lev * 4 + q] = m
            return tuple(t)

        return plsc.parallel_loop(0, nvec, unroll=2, carry=tops)(vec_step)

    def load_tops(i):
        return tuple(tops_buf[i + q, k] for k in range(5) for q in range(4))

    def store_tops(i, tops):
        for k in range(5):
            for q in range(4):
                tops_buf[i + q, k] = tops[k * 4 + q]

    def group_step(g, carry):
        grp = wid * _GPW + g

        for i in range(8):
            for k in range(5):
                tops_buf[i, k] = neg
        chunk_copy(grp, 0, 0, sem0).start()
        pltpu.make_async_copy(tail_hbm.at[pl.ds(grp * 8, 8)], tb.at[0], semt).start()

        def chunk_step(c, carry):
            par = lax.rem(c, 2)

            @pl.when(par == 0)
            def _():
                chunk_copy(grp, c, 0, sem0).wait()

            @pl.when(par == 1)
            def _():
                chunk_copy(grp, c, 1, sem1).wait()

            npar = lax.rem(c + 1, 2)

            @pl.when((c + 1 < _NCH) & (npar == 0))
            def _():
                chunk_copy(grp, c + 1, 0, sem0).start()

            @pl.when((c + 1 < _NCH) & (npar == 1))
            def _():
                chunk_copy(grp, c + 1, 1, sem1).start()

            for i in range(0, 8, 4):
                tops = load_tops(i)
                tops = scan_quad(cb, par, i, _CVEC, tops)
                store_tops(i, tops)
            return carry

        lax.fori_loop(0, _NCH, chunk_step, jnp.int32(0))

        pltpu.make_async_copy(tail_hbm.at[pl.ds(grp * 8, 8)], tb.at[0], semt).wait()
        for i in range(0, 8, 4):
            tops = load_tops(i)
            tops = scan_quad(tb, 0, i, _TVEC, tops)
            for k in range(5):
                for q in range(4):
                    out_buf[i + q, pl.ds(k * _L, _L)] = tops[k * 4 + q]
            for k in range(5, 8):
                for q in range(4):
                    out_buf[i + q, pl.ds(k * _L, _L)] = neg
        pltpu.sync_copy(out_buf, cand_hbm.at[pl.ds(grp * 8, 8)])
        return carry

    lax.fori_loop(0, _GPW, group_step, jnp.int32(0))


def _tc_body(cand_ref, out_ref):
    x = cand_ref[...]
    col = lax.broadcasted_iota(jnp.int32, x.shape, 1)
    s = jnp.zeros((x.shape[0], 1), jnp.float32)
    for _ in range(_TOP_K):
        m = jnp.max(x, axis=1, keepdims=True)
        idx = jnp.min(jnp.where(x == m, col, jnp.int32(_CAND)),
                      axis=1, keepdims=True)
        x = jnp.where(col == idx, jnp.float32(_NEG), x)
        s = s + m
    loss = jnp.maximum(jnp.float32(_MIN_ACT) - s * jnp.float32(1.0 / _TOP_K),
                       jnp.float32(0.0))
    out_ref[...] = jnp.reshape(jnp.sum(loss), (1, 1))


def _tc_stream_body(x_ref, out_ref):
    neg = jnp.full((_TCB, 128), _NEG, jnp.float32)
    nchunk = _NFULL // 128

    def chunk_step(c, tops):
        t0, t1, t2, t3, t4 = tops
        x = x_ref[:, pl.ds(c * 128, 128)]
        m0 = jnp.maximum(t0, x)
        cx = jnp.minimum(t0, x)
        m1 = jnp.maximum(t1, cx)
        cx = jnp.minimum(t1, cx)
        m2 = jnp.maximum(t2, cx)
        cx = jnp.minimum(t2, cx)
        m3 = jnp.maximum(t3, cx)
        cx = jnp.minimum(t3, cx)
        m4 = jnp.maximum(t4, cx)
        return (m0, m1, m2, m3, m4)

    tops = lax.fori_loop(0, nchunk, chunk_step, (neg, neg, neg, neg, neg))
    xt = x_ref[:, pl.ds(_NFULL, _N - _NFULL)]
    xt = jnp.concatenate(
        [xt, jnp.full((_TCB, 128 - (_N - _NFULL)), _NEG, jnp.float32)], axis=1)
    t0, t1, t2, t3, t4 = tops
    m0 = jnp.maximum(t0, xt)
    cx = jnp.minimum(t0, xt)
    m1 = jnp.maximum(t1, cx)
    cx = jnp.minimum(t1, cx)
    m2 = jnp.maximum(t2, cx)
    cx = jnp.minimum(t2, cx)
    m3 = jnp.maximum(t3, cx)
    cx = jnp.minimum(t3, cx)
    m4 = jnp.maximum(t4, cx)

    cat = jnp.concatenate([m0, m1, m2, m3, m4], axis=1)
    col = lax.broadcasted_iota(jnp.int32, cat.shape, 1)
    s = jnp.zeros((_TCB, 1), jnp.float32)
    for _ in range(_TOP_K):
        m = jnp.max(cat, axis=1, keepdims=True)
        idx = jnp.min(jnp.where(cat == m, col, jnp.int32(640)),
                      axis=1, keepdims=True)
        cat = jnp.where(col == idx, jnp.float32(_NEG), cat)
        s = s + m
    loss = jnp.maximum(jnp.float32(_MIN_ACT) - s * jnp.float32(1.0 / _TOP_K),
                       jnp.float32(0.0))
    out_ref[...] = jnp.reshape(jnp.sum(loss), (1, 1, 1))


@jax.jit
def kernel(sparse_repr):
    tail = jnp.concatenate(
        [sparse_repr[:_SCR, _MAIN:],
         jnp.full((_SCR, _TW - (_N - _MAIN)), _NEG, jnp.float32)], axis=1)
    mesh = plsc.VectorSubcoreMesh(core_axis_name="c", subcore_axis_name="s")
    cand = pl.kernel(
        _sc_body,
        out_type=jax.ShapeDtypeStruct((_SCR, _CAND), jnp.float32),
        mesh=mesh,
        scratch_types=[
            pltpu.VMEM((2, 8, _CW), jnp.float32),
            pltpu.VMEM((1, 8, _TW), jnp.float32),
            pltpu.VMEM((8, 5, _L), jnp.float32),
            pltpu.VMEM((8, _CAND), jnp.float32),
            pltpu.SemaphoreType.DMA,
            pltpu.SemaphoreType.DMA,
            pltpu.SemaphoreType.DMA,
        ],
    )(sparse_repr, tail)
    tc_part = pl.pallas_call(
        _tc_stream_body,
        grid=(_TCR // _TCB,),
        in_specs=[pl.BlockSpec((_TCB, _N), lambda i: (i + _SCR // _TCB, 0))],
        out_specs=pl.BlockSpec((1, 1, 1), lambda i: (i, 0, 0)),
        out_shape=jax.ShapeDtypeStruct((_TCR // _TCB, 1, 1), jnp.float32),
    )(sparse_repr)
    sc_part = pl.pallas_call(
        _tc_body,
        out_shape=jax.ShapeDtypeStruct((1, 1), jnp.float32),
        in_specs=[pl.BlockSpec(memory_space=pltpu.VMEM)],
        out_specs=pl.BlockSpec(memory_space=pltpu.VMEM),
    )(cand)
    return (sc_part[0, 0] + jnp.sum(tc_part)) * jnp.float32(1.0 / _R)


# speedup vs baseline: 1.5720x; 1.5720x over previous
import jax
import jax.numpy as jnp
from jax import lax
from jax.experimental import pallas as pl
from jax.experimental.pallas import tpu as pltpu
from jax.experimental.pallas import tpu_sc as plsc

_R = 1024
_N = 100000
_L = 16
_NC = 2
_NS = 16
_NW = _NC * _NS
_SCR = 512
_TCR = _R - _SCR
_GPW = _SCR // (_NW * 8)
_TCB = 32
_TCC = 512
_NFULL = _N // _TCC * _TCC

_CW = 3200
_NCH = 31
_MAIN = _NCH * _CW
_TW = 1024
_CVEC = _CW // _L
_TVEC = _TW // _L
_CAND = 128
_TOP_K = 5
_MIN_ACT = 0.5
_NEG = float(-jnp.inf)


def _sc_body(x_hbm, tail_hbm, cand_hbm, cb, tb, tops_buf, out_buf,
             sem0, sem1, semt):
    cid = lax.axis_index("c")
    sid = lax.axis_index("s")
    wid = cid * _NS + sid
    neg = jnp.full((_L,), _NEG, jnp.float32)
    sems = (sem0, sem1)

    def chunk_copy(grp, c, par, sem):
        return pltpu.make_async_copy(
            x_hbm.at[pl.ds(grp * 8, 8), pl.ds(c * _CW, _CW)],
            cb.at[par], sem)

    def scan_quad(buf, par, i, nvec, tops):
        def vec_step(j, t):
            t = list(t)
            xs = [buf[par, i + q, pl.ds(j * _L, _L)] for q in range(4)]
            for lev in range(5):
                for q in range(4):
                    m = jnp.maximum(t[lev * 4 + q], xs[q])
                    if lev < 4:
                        xs[q] = jnp.minimum(t[lev * 4 + q], xs[q])
                    t[lev * 4 + q] = m
            return tuple(t)

        return plsc.parallel_loop(0, nvec, unroll=2, carry=tops)(vec_step)

    def load_tops(i):
        return tuple(tops_buf[i + q, k] for k in range(5) for q in range(4))

    def store_tops(i, tops):
        for k in range(5):
            for q in range(4):
                tops_buf[i + q, k] = tops[k * 4 + q]

    def group_step(g, carry):
        grp = wid * _GPW + g

        for i in range(8):
            for k in range(5):
                tops_buf[i, k] = neg
        chunk_copy(grp, 0, 0, sem0).start()
        pltpu.make_async_copy(tail_hbm.at[pl.ds(grp * 8, 8)], tb.at[0], semt).start()

        def chunk_step(c, carry):
            par = lax.rem(c, 2)

            @pl.when(par == 0)
            def _():
                chunk_copy(grp, c, 0, sem0).wait()

            @pl.when(par == 1)
            def _():
                chunk_copy(grp, c, 1, sem1).wait()

            npar = lax.rem(c + 1, 2)

            @pl.when((c + 1 < _NCH) & (npar == 0))
            def _():
                chunk_copy(grp, c + 1, 0, sem0).start()

            @pl.when((c + 1 < _NCH) & (npar == 1))
            def _():
                chunk_copy(grp, c + 1, 1, sem1).start()

            for i in range(0, 8, 4):
                tops = load_tops(i)
                tops = scan_quad(cb, par, i, _CVEC, tops)
                store_tops(i, tops)
            return carry

        lax.fori_loop(0, _NCH, chunk_step, jnp.int32(0))

        pltpu.make_async_copy(tail_hbm.at[pl.ds(grp * 8, 8)], tb.at[0], semt).wait()
        for i in range(0, 8, 4):
            tops = load_tops(i)
            tops = scan_quad(tb, 0, i, _TVEC, tops)
            for k in range(5):
                for q in range(4):
                    out_buf[i + q, pl.ds(k * _L, _L)] = tops[k * 4 + q]
            for k in range(5, 8):
                for q in range(4):
                    out_buf[i + q, pl.ds(k * _L, _L)] = neg
        pltpu.sync_copy(out_buf, cand_hbm.at[pl.ds(grp * 8, 8)])
        return carry

    lax.fori_loop(0, _GPW, group_step, jnp.int32(0))


def _tc_body(cand_ref, out_ref):
    x = cand_ref[...]
    col = lax.broadcasted_iota(jnp.int32, x.shape, 1)
    s = jnp.zeros((x.shape[0], 1), jnp.float32)
    for _ in range(_TOP_K):
        m = jnp.max(x, axis=1, keepdims=True)
        idx = jnp.min(jnp.where(x == m, col, jnp.int32(_CAND)),
                      axis=1, keepdims=True)
        x = jnp.where(col == idx, jnp.float32(_NEG), x)
        s = s + m
    loss = jnp.maximum(jnp.float32(_MIN_ACT) - s * jnp.float32(1.0 / _TOP_K),
                       jnp.float32(0.0))
    out_ref[...] = jnp.reshape(jnp.sum(loss), (1, 1))


def _tc_stream_body(x_ref, out_ref):
    neg = jnp.full((_TCB, _TCC), _NEG, jnp.float32)
    nchunk = _NFULL // _TCC

    def chunk_step(c, tops):
        t0, t1, t2, t3, t4 = tops
        x = x_ref[:, pl.ds(c * _TCC, _TCC)]
        m0 = jnp.maximum(t0, x)
        cx = jnp.minimum(t0, x)
        m1 = jnp.maximum(t1, cx)
        cx = jnp.minimum(t1, cx)
        m2 = jnp.maximum(t2, cx)
        cx = jnp.minimum(t2, cx)
        m3 = jnp.maximum(t3, cx)
        cx = jnp.minimum(t3, cx)
        m4 = jnp.maximum(t4, cx)
        return (m0, m1, m2, m3, m4)

    tops = lax.fori_loop(0, nchunk, chunk_step, (neg, neg, neg, neg, neg))
    xt = x_ref[:, pl.ds(_NFULL, _N - _NFULL)]
    xt = jnp.concatenate(
        [xt, jnp.full((_TCB, _TCC - (_N - _NFULL)), _NEG, jnp.float32)],
        axis=1)
    t0, t1, t2, t3, t4 = tops
    m0 = jnp.maximum(t0, xt)
    cx = jnp.minimum(t0, xt)
    m1 = jnp.maximum(t1, cx)
    cx = jnp.minimum(t1, cx)
    m2 = jnp.maximum(t2, cx)
    cx = jnp.minimum(t2, cx)
    m3 = jnp.maximum(t3, cx)
    cx = jnp.minimum(t3, cx)
    m4 = jnp.maximum(t4, cx)

    cat = jnp.concatenate([m0, m1, m2, m3, m4], axis=1)
    col = lax.broadcasted_iota(jnp.int32, cat.shape, 1)
    s = jnp.zeros((_TCB, 1), jnp.float32)
    for _ in range(_TOP_K):
        m = jnp.max(cat, axis=1, keepdims=True)
        idx = jnp.min(jnp.where(cat == m, col, jnp.int32(5 * _TCC)),
                      axis=1, keepdims=True)
        cat = jnp.where(col == idx, jnp.float32(_NEG), cat)
        s = s + m
    loss = jnp.maximum(jnp.float32(_MIN_ACT) - s * jnp.float32(1.0 / _TOP_K),
                       jnp.float32(0.0))
    out_ref[...] = jnp.reshape(jnp.sum(loss), (1, 1, 1))


@jax.jit
def kernel(sparse_repr):
    tail = jnp.concatenate(
        [sparse_repr[:_SCR, _MAIN:],
         jnp.full((_SCR, _TW - (_N - _MAIN)), _NEG, jnp.float32)], axis=1)
    mesh = plsc.VectorSubcoreMesh(core_axis_name="c", subcore_axis_name="s")
    cand = pl.kernel(
        _sc_body,
        out_type=jax.ShapeDtypeStruct((_SCR, _CAND), jnp.float32),
        mesh=mesh,
        scratch_types=[
            pltpu.VMEM((2, 8, _CW), jnp.float32),
            pltpu.VMEM((1, 8, _TW), jnp.float32),
            pltpu.VMEM((8, 5, _L), jnp.float32),
            pltpu.VMEM((8, _CAND), jnp.float32),
            pltpu.SemaphoreType.DMA,
            pltpu.SemaphoreType.DMA,
            pltpu.SemaphoreType.DMA,
        ],
    )(sparse_repr, tail)
    tc_part = pl.pallas_call(
        _tc_stream_body,
        grid=(_TCR // _TCB,),
        in_specs=[pl.BlockSpec((_TCB, _N), lambda i: (i + _SCR // _TCB, 0))],
        out_specs=pl.BlockSpec((1, 1, 1), lambda i: (i, 0, 0)),
        out_shape=jax.ShapeDtypeStruct((_TCR // _TCB, 1, 1), jnp.float32),
    )(sparse_repr)
    sc_part = pl.pallas_call(
        _tc_body,
        out_shape=jax.ShapeDtypeStruct((1, 1), jnp.float32),
        in_specs=[pl.BlockSpec(memory_space=pltpu.VMEM)],
        out_specs=pl.BlockSpec(memory_space=pltpu.VMEM),
    )(cand)
    return (sc_part[0, 0] + jnp.sum(tc_part)) * jnp.float32(1.0 / _R)
